# baseline (device time: 23955 ns/iter reference)
import jax
import jax.numpy as jnp
from jax import lax
from jax.experimental import pallas as pl
from jax.experimental.pallas import tpu as pltpu

CHUNK_ROWS = (256, 192, 160, 128, 96, 80, 64, 48)
N_CHUNKS = len(CHUNK_ROWS)
CHUNK_OFF = tuple(sum(CHUNK_ROWS[:i]) for i in range(N_CHUNKS))


def kernel(x):
    m, n = x.shape
    half = m // 2
    assert sum(CHUNK_ROWS) == half

    def body(x_ref, out_ref, send_a, recv_a,
             send_a_sems, recv_a_sems, send_b_sems, recv_b_sems):
        my_x = lax.axis_index("x")
        my_y = lax.axis_index("y")
        y_nbr = (my_x, 1 - my_y)
        x_nbr = (1 - my_x, my_y)

        my_base = my_x * half

        barrier_sem = pltpu.get_barrier_semaphore()
        for nbr in (y_nbr, x_nbr):
            pl.semaphore_signal(
                barrier_sem, inc=1,
                device_id=nbr, device_id_type=pl.DeviceIdType.MESH,
            )
        send_a[...] = x_ref[pl.ds(my_base, half), :].astype(jnp.bfloat16)
        pl.semaphore_wait(barrier_sem, 2)

        rdma_a = []
        for c in range(N_CHUNKS):
            sl = pl.ds(CHUNK_OFF[c], CHUNK_ROWS[c])
            r = pltpu.make_async_remote_copy(
                src_ref=send_a.at[sl],
                dst_ref=recv_a.at[sl],
                send_sem=send_a_sems.at[c],
                recv_sem=recv_a_sems.at[c],
                device_id=y_nbr,
                device_id_type=pl.DeviceIdType.MESH,
            )
            r.start()
            rdma_a.append(r)

        rdma_b = []
        for c in range(N_CHUNKS):
            osl = pl.ds(my_base + CHUNK_OFF[c], CHUNK_ROWS[c])
            rdma_a[c].wait_recv()
            out_ref[osl, :] = (
                x_ref[osl, :]
                + recv_a[pl.ds(CHUNK_OFF[c], CHUNK_ROWS[c]), :]
                .astype(jnp.float32)
            ).astype(jnp.bfloat16)
            r = pltpu.make_async_remote_copy(
                src_ref=out_ref.at[osl],
                dst_ref=out_ref.at[osl],
                send_sem=send_b_sems.at[c],
                recv_sem=recv_b_sems.at[c],
                device_id=x_nbr,
                device_id_type=pl.DeviceIdType.MESH,
            )
            r.start()
            rdma_b.append(r)

        for c in range(N_CHUNKS):
            rdma_b[c].wait_recv()

        for c in range(N_CHUNKS):
            rdma_a[c].wait_send()
            rdma_b[c].wait_send()

    return pl.pallas_call(
        body,
        out_shape=jax.ShapeDtypeStruct((m, n), jnp.bfloat16),
        in_specs=[pl.BlockSpec(memory_space=pltpu.VMEM)],
        out_specs=pl.BlockSpec(memory_space=pltpu.VMEM),
        scratch_shapes=[
            pltpu.VMEM((half, n), jnp.bfloat16),
            pltpu.VMEM((half, n), jnp.bfloat16),
            pltpu.SemaphoreType.DMA((N_CHUNKS,)),
            pltpu.SemaphoreType.DMA((N_CHUNKS,)),
            pltpu.SemaphoreType.DMA((N_CHUNKS,)),
            pltpu.SemaphoreType.DMA((N_CHUNKS,)),
        ],
        compiler_params=pltpu.CompilerParams(collective_id=0),
    )(x)


# device time: 22540 ns/iter; 1.0628x vs baseline; 1.0628x over previous
import jax
import jax.numpy as jnp
from jax import lax
from jax.experimental import pallas as pl
from jax.experimental.pallas import tpu as pltpu

CHUNK_ROWS = (128, 128, 128, 128, 128, 128, 128, 64, 64)
N_CHUNKS = len(CHUNK_ROWS)
CHUNK_OFF = tuple(sum(CHUNK_ROWS[:i]) for i in range(N_CHUNKS))


def kernel(x):
    m, n = x.shape
    half = m // 2
    assert sum(CHUNK_ROWS) == half

    def body(x_ref, out_ref, send_a, recv_a,
             send_a_sems, recv_a_sems, send_b_sems, recv_b_sems):
        my_x = lax.axis_index("x")
        my_y = lax.axis_index("y")
        y_nbr = (my_x, 1 - my_y)
        x_nbr = (1 - my_x, my_y)

        my_base = my_x * half

        barrier_sem = pltpu.get_barrier_semaphore()
        for nbr in (y_nbr, x_nbr):
            pl.semaphore_signal(
                barrier_sem, inc=1,
                device_id=nbr, device_id_type=pl.DeviceIdType.MESH,
            )
        send_a[...] = x_ref[pl.ds(my_base, half), :].astype(jnp.bfloat16)
        pl.semaphore_wait(barrier_sem, 2)

        rdma_a = []
        for c in range(N_CHUNKS):
            sl = pl.ds(CHUNK_OFF[c], CHUNK_ROWS[c])
            r = pltpu.make_async_remote_copy(
                src_ref=send_a.at[sl],
                dst_ref=recv_a.at[sl],
                send_sem=send_a_sems.at[c],
                recv_sem=recv_a_sems.at[c],
                device_id=y_nbr,
                device_id_type=pl.DeviceIdType.MESH,
            )
            r.start()
            rdma_a.append(r)

        rdma_b = []
        for c in range(N_CHUNKS):
            osl = pl.ds(my_base + CHUNK_OFF[c], CHUNK_ROWS[c])
            rdma_a[c].wait_recv()
            out_ref[osl, :] = (
                x_ref[osl, :]
                + recv_a[pl.ds(CHUNK_OFF[c], CHUNK_ROWS[c]), :]
                .astype(jnp.float32)
            ).astype(jnp.bfloat16)
            r = pltpu.make_async_remote_copy(
                src_ref=out_ref.at[osl],
                dst_ref=out_ref.at[osl],
                send_sem=send_b_sems.at[c],
                recv_sem=recv_b_sems.at[c],
                device_id=x_nbr,
                device_id_type=pl.DeviceIdType.MESH,
            )
            r.start()
            rdma_b.append(r)

        for c in range(N_CHUNKS):
            rdma_b[c].wait_recv()

        for c in range(N_CHUNKS):
            rdma_a[c].wait_send()
            rdma_b[c].wait_send()

    return pl.pallas_call(
        body,
        out_shape=jax.ShapeDtypeStruct((m, n), jnp.bfloat16),
        in_specs=[pl.BlockSpec(memory_space=pltpu.VMEM)],
        out_specs=pl.BlockSpec(memory_space=pltpu.VMEM),
        scratch_shapes=[
            pltpu.VMEM((half, n), jnp.bfloat16),
            pltpu.VMEM((half, n), jnp.bfloat16),
            pltpu.SemaphoreType.DMA((N_CHUNKS,)),
            pltpu.SemaphoreType.DMA((N_CHUNKS,)),
            pltpu.SemaphoreType.DMA((N_CHUNKS,)),
            pltpu.SemaphoreType.DMA((N_CHUNKS,)),
        ],
        compiler_params=pltpu.CompilerParams(collective_id=0),
    )(x)
